# Initial kernel scaffold; baseline (speedup 1.0000x reference)
import jax
import jax.numpy as jnp
from jax import lax
from jax.experimental import pallas as pl
from jax.experimental.pallas import tpu as pltpu
from jax.experimental.pallas import tpu_sc as plsc

N = 100000
E = 1600000
F_IN = 128
H = 32
C = 10
G = 64
HH = H // 2

RB = 2000
GRID = N // RB

_PE = 784 * 2048
_ROWS = _PE // 128
_TROWS = _ROWS // 16
_GU = 16
_NG = _TROWS // _GU
_NPAD = N + 8
_FR = N // 16
_ZR = 1250

_F32 = jnp.float32


def _sc_body(hlo, hhi, src, dst, alo, ahi, acc, sidx, didx, rows, zbuf, gsem):
    cid = lax.axis_index("c")
    sid = lax.axis_index("s")

    def _zb(i, c):
        zbuf[i, :] = jnp.zeros((16,), _F32)
        return c

    lax.fori_loop(0, _ZR, _zb, 0)
    zbase = sid * _FR
    for t in range(5):
        pltpu.sync_copy(zbuf, acc.at[pl.ds(zbase + t * _ZR, _ZR)])
    plsc.subcore_barrier()

    def _run(hsrc, aout):
        row_base = sid * _TROWS

        def _grp(g, c):
            r0 = row_base + g * _GU
            pltpu.sync_copy(src.at[pl.ds(r0, _GU)], sidx)
            pltpu.sync_copy(dst.at[pl.ds(r0, _GU)], didx)
            descs = [
                pltpu.async_copy(hsrc.at[sidx.at[j]], rows.at[j], gsem)
                for j in range(_GU)
            ]
            for d in descs:
                d.wait()
            for j in range(_GU):
                pltpu.sync_copy(rows.at[j], acc.at[didx.at[j]], add=True)
            return c

        lax.fori_loop(0, _NG, _grp, 0)
        plsc.subcore_barrier()
        fr0 = sid * _FR
        pltpu.sync_copy(acc.at[pl.ds(fr0, _FR)], aout.at[pl.ds(fr0, _FR)])

    @pl.when(cid == 0)
    def _():
        _run(hlo, alo)

    @pl.when(cid == 1)
    def _():
        _run(hhi, ahi)


def _sc_aggregate(h_lo, h_hi, src2d, dst2d):
    mesh = plsc.VectorSubcoreMesh(core_axis_name="c", subcore_axis_name="s")
    kern = pl.kernel(
        _sc_body,
        out_type=(jax.ShapeDtypeStruct((N, HH), _F32),
                  jax.ShapeDtypeStruct((N, HH), _F32)),
        mesh=mesh,
        scratch_types=[
            pltpu.VMEM_SHARED((_NPAD, HH), _F32),
            pltpu.VMEM((_GU, 128), jnp.int32),
            pltpu.VMEM((_GU, 128), jnp.int32),
            pltpu.VMEM((_GU, 128, HH), _F32),
            pltpu.VMEM((_ZR, HH), _F32),
            pltpu.SemaphoreType.DMA,
        ],
    )
    return kern(h_lo, h_hi, src2d, dst2d)


def _lin1_body(x_ref, w_ref, b_ref, lo_ref, hi_ref):
    h = jnp.dot(x_ref[...], w_ref[...], preferred_element_type=_F32)
    h = jnp.maximum(h + b_ref[...], 0.0)
    lo_ref[...] = h[:, :HH]
    hi_ref[...] = h[:, HH:]


def _lin1(x, w, b):
    return pl.pallas_call(
        _lin1_body,
        grid=(GRID,),
        in_specs=[
            pl.BlockSpec((RB, F_IN), lambda i: (i, 0)),
            pl.BlockSpec((F_IN, H), lambda i: (0, 0)),
            pl.BlockSpec((1, H), lambda i: (0, 0)),
        ],
        out_specs=[
            pl.BlockSpec((RB, HH), lambda i: (i, 0)),
            pl.BlockSpec((RB, HH), lambda i: (i, 0)),
        ],
        out_shape=[jax.ShapeDtypeStruct((N, HH), _F32)] * 2,
    )(x, w, b)


def _mlp_core(lo_ref, hi_ref, alo_ref, ahi_ref, w1_ref, b1_ref, w2_ref, b2_ref):
    mlo = lo_ref[...] + alo_ref[...]
    mhi = hi_ref[...] + ahi_ref[...]
    w1 = w1_ref[...]
    t = (jnp.dot(mlo, w1[:HH, :], preferred_element_type=_F32)
         + jnp.dot(mhi, w1[HH:, :], preferred_element_type=_F32)
         + b1_ref[...])
    t = jnp.maximum(t, 0.0)
    o = jnp.dot(t, w2_ref[...], preferred_element_type=_F32) + b2_ref[...]
    return jnp.maximum(o, 0.0)


def _mlp_body(lo_ref, hi_ref, alo_ref, ahi_ref, w1_ref, b1_ref, w2_ref,
              b2_ref, olo_ref, ohi_ref):
    o = _mlp_core(lo_ref, hi_ref, alo_ref, ahi_ref, w1_ref, b1_ref,
                  w2_ref, b2_ref)
    olo_ref[...] = o[:, :HH]
    ohi_ref[...] = o[:, HH:]


def _mlp(lo, hi, alo, ahi, w1, b1, w2, b2):
    node_spec = pl.BlockSpec((RB, HH), lambda i: (i, 0))
    return pl.pallas_call(
        _mlp_body,
        grid=(GRID,),
        in_specs=[
            node_spec, node_spec, node_spec, node_spec,
            pl.BlockSpec((H, H), lambda i: (0, 0)),
            pl.BlockSpec((1, H), lambda i: (0, 0)),
            pl.BlockSpec((H, H), lambda i: (0, 0)),
            pl.BlockSpec((1, H), lambda i: (0, 0)),
        ],
        out_specs=[node_spec, node_spec],
        out_shape=[jax.ShapeDtypeStruct((N, HH), _F32)] * 2,
    )(lo, hi, alo, ahi, w1, b1, w2, b2)


def _mlp_stats_body(lo_ref, hi_ref, alo_ref, ahi_ref, w1_ref, b1_ref, w2_ref,
                    b2_ref, bat_ref, sq_ref, p_ref, cnt_ref):
    o = _mlp_core(lo_ref, hi_ref, alo_ref, ahi_ref, w1_ref, b1_ref,
                  w2_ref, b2_ref)

    @pl.when(pl.program_id(0) == 0)
    def _():
        sq_ref[...] = jnp.zeros_like(sq_ref)
        p_ref[...] = jnp.zeros_like(p_ref)
        cnt_ref[...] = jnp.zeros_like(cnt_ref)

    ids = bat_ref[...].reshape(1, RB)
    onehot = (lax.broadcasted_iota(jnp.int32, (G, 1), 0) == ids).astype(_F32)
    sq_ref[0:1, :] += jnp.sum(o, axis=0, keepdims=True)
    sq_ref[1:2, :] += jnp.sum(o * o, axis=0, keepdims=True)
    p_ref[...] += jnp.dot(onehot, o, preferred_element_type=_F32)
    cnt_ref[...] += jnp.sum(onehot, axis=1, keepdims=True)


def _mlp_stats(lo, hi, alo, ahi, w1, b1, w2, b2, bat):
    node_spec = pl.BlockSpec((RB, HH), lambda i: (i, 0))
    return pl.pallas_call(
        _mlp_stats_body,
        grid=(GRID,),
        in_specs=[
            node_spec, node_spec, node_spec, node_spec,
            pl.BlockSpec((H, H), lambda i: (0, 0)),
            pl.BlockSpec((1, H), lambda i: (0, 0)),
            pl.BlockSpec((H, H), lambda i: (0, 0)),
            pl.BlockSpec((1, H), lambda i: (0, 0)),
            pl.BlockSpec((1, 1, RB), lambda i: (i, 0, 0)),
        ],
        out_specs=[
            pl.BlockSpec((2, H), lambda i: (0, 0)),
            pl.BlockSpec((G, H), lambda i: (0, 0)),
            pl.BlockSpec((G, 1), lambda i: (0, 0)),
        ],
        out_shape=[
            jax.ShapeDtypeStruct((2, H), _F32),
            jax.ShapeDtypeStruct((G, H), _F32),
            jax.ShapeDtypeStruct((G, 1), _F32),
        ],
    )(lo, hi, alo, ahi, w1, b1, w2, b2, bat)


def _head_body(sq_ref, p_ref, cnt_ref, gam_ref, bet_ref, w_ref, b_ref,
               out_ref):
    inv_n = _F32(1.0 / N)
    mu = sq_ref[0:1, :] * inv_n
    var = sq_ref[1:2, :] * inv_n - mu * mu
    scale = lax.rsqrt(var + 1e-5) * gam_ref[...]
    c = cnt_ref[...]
    pooled = p_ref[...] / jnp.maximum(c, 1.0)
    normed = (pooled - mu) * scale + bet_ref[...]
    normed = jnp.where(c > 0.0, normed, 0.0)
    logits = jnp.dot(normed, w_ref[...], preferred_element_type=_F32)
    logits = logits + b_ref[...]
    m = jnp.max(logits, axis=1, keepdims=True)
    sh = logits - m
    out_ref[...] = sh - jnp.log(jnp.sum(jnp.exp(sh), axis=1, keepdims=True))


def _head(sq, p, cnt, gam, bet, w, b):
    return pl.pallas_call(
        _head_body,
        out_shape=jax.ShapeDtypeStruct((G, C), _F32),
    )(sq, p, cnt, gam, bet, w, b)


def kernel(x, edge_index, batch, lin1_w, lin1_b, g0_w1, g0_b1, g0_w2, g0_b2,
           g1_w1, g1_b1, g1_w2, g1_b2, g2_w1, g2_b1, g2_w2, g2_b2,
           bn_gamma, bn_beta, lin2_w, lin2_b):
    pad = _PE - E
    src = jnp.concatenate(
        [edge_index[0].astype(jnp.int32), jnp.zeros((pad,), jnp.int32)])
    dst = jnp.concatenate(
        [edge_index[1].astype(jnp.int32), jnp.full((pad,), N, jnp.int32)])
    src2d = src.reshape(_ROWS, 128)
    dst2d = dst.reshape(_ROWS, 128)
    bat = batch.astype(jnp.int32).reshape(GRID, 1, RB)

    hlo, hhi = _lin1(x, lin1_w, lin1_b.reshape(1, H))

    layers = ((g0_w1, g0_b1, g0_w2, g0_b2),
              (g1_w1, g1_b1, g1_w2, g1_b2))
    for w1, b1, w2, b2 in layers:
        alo, ahi = _sc_aggregate(hlo, hhi, src2d, dst2d)
        hlo, hhi = _mlp(hlo, hhi, alo, ahi, w1, b1.reshape(1, H),
                        w2, b2.reshape(1, H))

    alo, ahi = _sc_aggregate(hlo, hhi, src2d, dst2d)
    sq, p, cnt = _mlp_stats(hlo, hhi, alo, ahi, g2_w1, g2_b1.reshape(1, H),
                            g2_w2, g2_b2.reshape(1, H), bat)

    return _head(sq, p, cnt, bn_gamma.reshape(1, H), bn_beta.reshape(1, H),
                 lin2_w, lin2_b.reshape(1, C))


# speedup vs baseline: 9.6800x; 9.6800x over previous
import jax
import jax.numpy as jnp
import numpy as np
from jax import lax
from jax.experimental import pallas as pl
from jax.experimental.pallas import tpu as pltpu
from jax.experimental.pallas import tpu_sc as plsc

N = 100000
E = 1600000
F_IN = 128
H = 32
C = 10
G = 64
HH = H // 2

RB = 2000
GRID = N // RB

_PE = 784 * 2048
_ROWS = _PE // 128
_TROWS = _ROWS // 16
_GU = 8
_NG = _TROWS // _GU
_FR = 6256
_NPAD = 16 * _FR
_FR_LAST = N - 15 * _FR

_F32 = jnp.float32

_I0 = np.int32(0)


def _im_row(i):
    return (i, _I0)


def _im_fix(i):
    return (_I0, _I0)


def _im_bat(i):
    return (i, _I0, _I0)


def _sc_body(hlo, hhi, src, dst, alo, ahi, acc, sidx, didx, rows, gsem):
    cid = lax.axis_index("c")
    sid = lax.axis_index("s")

    def _zb(i, c):
        rows[jnp.int32(0), i, :] = jnp.zeros((16,), _F32)
        return c

    lax.fori_loop(jnp.int32(0), jnp.int32(128), _zb, jnp.int32(0))
    z0 = rows.at[jnp.int32(0)]
    zbase = sid * jnp.int32(_FR)
    for t in range(48):
        pltpu.sync_copy(z0, acc.at[pl.ds(zbase + t * 128, 128)])
    pltpu.sync_copy(z0.at[pl.ds(0, 112)],
                    acc.at[pl.ds(zbase + 48 * 128, 112)])
    plsc.subcore_barrier()

    def _run(hsrc, aout):
        row_base = sid * jnp.int32(_TROWS)

        def _grp(g, c):
            r0 = row_base + g * jnp.int32(_GU)
            pltpu.sync_copy(src.at[pl.ds(r0, _GU)], sidx)
            pltpu.sync_copy(dst.at[pl.ds(r0, _GU)], didx)
            descs = [
                pltpu.async_copy(hsrc.at[sidx.at[jnp.int32(j)]],
                                 rows.at[jnp.int32(j)], gsem)
                for j in range(_GU)
            ]
            for d in descs:
                d.wait()
            for j in range(_GU):
                pltpu.sync_copy(rows.at[jnp.int32(j)],
                                acc.at[didx.at[jnp.int32(j)]], add=True)
            return c

        lax.fori_loop(jnp.int32(0), jnp.int32(_NG), _grp, jnp.int32(0))
        plsc.subcore_barrier()
        fr0 = sid * jnp.int32(_FR)

        @pl.when(sid < 15)
        def _():
            pltpu.sync_copy(acc.at[pl.ds(fr0, _FR)], aout.at[pl.ds(fr0, _FR)])

        @pl.when(sid == 15)
        def _():
            pltpu.sync_copy(acc.at[pl.ds(fr0, _FR_LAST)],
                            aout.at[pl.ds(fr0, _FR_LAST)])

    @pl.when(cid == 0)
    def _():
        _run(hlo, alo)

    @pl.when(cid == 1)
    def _():
        _run(hhi, ahi)


def _sc_aggregate(h_lo, h_hi, src2d, dst2d):
    mesh = plsc.VectorSubcoreMesh(core_axis_name="c", subcore_axis_name="s")
    kern = pl.kernel(
        _sc_body,
        out_type=(jax.ShapeDtypeStruct((N, HH), _F32),
                  jax.ShapeDtypeStruct((N, HH), _F32)),
        mesh=mesh,
        scratch_types=[
            pltpu.VMEM_SHARED((_NPAD, HH), _F32),
            pltpu.VMEM((_GU, 128), jnp.int32),
            pltpu.VMEM((_GU, 128), jnp.int32),
            pltpu.VMEM((_GU, 128, HH), _F32),
            pltpu.SemaphoreType.DMA,
        ],
        compiler_params=pltpu.CompilerParams(use_tc_tiling_on_sc=False),
    )
    return kern(h_lo, h_hi, src2d, dst2d)


def _lin1_body(x_ref, w_ref, b_ref, lo_ref, hi_ref):
    h = jnp.dot(x_ref[...], w_ref[...], preferred_element_type=_F32)
    h = jnp.maximum(h + b_ref[...], 0.0)
    lo_ref[...] = h[:, :HH]
    hi_ref[...] = h[:, HH:]


def _lin1(x, w, b):
    return pl.pallas_call(
        _lin1_body,
        grid=(GRID,),
        in_specs=[
            pl.BlockSpec((RB, F_IN), _im_row),
            pl.BlockSpec((F_IN, H), _im_fix),
            pl.BlockSpec((1, H), _im_fix),
        ],
        out_specs=[
            pl.BlockSpec((RB, HH), _im_row),
            pl.BlockSpec((RB, HH), _im_row),
        ],
        out_shape=[jax.ShapeDtypeStruct((N, HH), _F32)] * 2,
    )(x, w, b)


def _mlp_core(lo_ref, hi_ref, alo_ref, ahi_ref, w1_ref, b1_ref, w2_ref, b2_ref):
    mlo = lo_ref[...] + alo_ref[...]
    mhi = hi_ref[...] + ahi_ref[...]
    w1 = w1_ref[...]
    t = (jnp.dot(mlo, w1[:HH, :], preferred_element_type=_F32)
         + jnp.dot(mhi, w1[HH:, :], preferred_element_type=_F32)
         + b1_ref[...])
    t = jnp.maximum(t, 0.0)
    o = jnp.dot(t, w2_ref[...], preferred_element_type=_F32) + b2_ref[...]
    return jnp.maximum(o, 0.0)


def _mlp_body(lo_ref, hi_ref, alo_ref, ahi_ref, w1_ref, b1_ref, w2_ref,
              b2_ref, olo_ref, ohi_ref):
    o = _mlp_core(lo_ref, hi_ref, alo_ref, ahi_ref, w1_ref, b1_ref,
                  w2_ref, b2_ref)
    olo_ref[...] = o[:, :HH]
    ohi_ref[...] = o[:, HH:]


def _mlp(lo, hi, alo, ahi, w1, b1, w2, b2):
    node_spec = pl.BlockSpec((RB, HH), _im_row)
    return pl.pallas_call(
        _mlp_body,
        grid=(GRID,),
        in_specs=[
            node_spec, node_spec, node_spec, node_spec,
            pl.BlockSpec((H, H), _im_fix),
            pl.BlockSpec((1, H), _im_fix),
            pl.BlockSpec((H, H), _im_fix),
            pl.BlockSpec((1, H), _im_fix),
        ],
        out_specs=[node_spec, node_spec],
        out_shape=[jax.ShapeDtypeStruct((N, HH), _F32)] * 2,
    )(lo, hi, alo, ahi, w1, b1, w2, b2)


def _mlp_stats_body(lo_ref, hi_ref, alo_ref, ahi_ref, w1_ref, b1_ref, w2_ref,
                    b2_ref, bat_ref, sq_ref, p_ref, cnt_ref):
    o = _mlp_core(lo_ref, hi_ref, alo_ref, ahi_ref, w1_ref, b1_ref,
                  w2_ref, b2_ref)

    @pl.when(pl.program_id(0) == 0)
    def _():
        sq_ref[...] = jnp.zeros_like(sq_ref)
        p_ref[...] = jnp.zeros_like(p_ref)
        cnt_ref[...] = jnp.zeros_like(cnt_ref)

    ids = bat_ref[...].reshape(1, RB)
    onehot = (lax.broadcasted_iota(jnp.int32, (G, 1), 0) == ids).astype(_F32)
    sq_ref[0:1, :] += jnp.sum(o, axis=0, keepdims=True)
    sq_ref[1:2, :] += jnp.sum(o * o, axis=0, keepdims=True)
    p_ref[...] += jnp.dot(onehot, o, preferred_element_type=_F32)
    cnt_ref[...] += jnp.sum(onehot, axis=1, keepdims=True)


def _mlp_stats(lo, hi, alo, ahi, w1, b1, w2, b2, bat):
    node_spec = pl.BlockSpec((RB, HH), _im_row)
    return pl.pallas_call(
        _mlp_stats_body,
        grid=(GRID,),
        in_specs=[
            node_spec, node_spec, node_spec, node_spec,
            pl.BlockSpec((H, H), _im_fix),
            pl.BlockSpec((1, H), _im_fix),
            pl.BlockSpec((H, H), _im_fix),
            pl.BlockSpec((1, H), _im_fix),
            pl.BlockSpec((1, 1, RB), _im_bat),
        ],
        out_specs=[
            pl.BlockSpec((2, H), _im_fix),
            pl.BlockSpec((G, H), _im_fix),
            pl.BlockSpec((G, 1), _im_fix),
        ],
        out_shape=[
            jax.ShapeDtypeStruct((2, H), _F32),
            jax.ShapeDtypeStruct((G, H), _F32),
            jax.ShapeDtypeStruct((G, 1), _F32),
        ],
    )(lo, hi, alo, ahi, w1, b1, w2, b2, bat)


def _head_body(sq_ref, p_ref, cnt_ref, gam_ref, bet_ref, w_ref, b_ref,
               out_ref):
    inv_n = _F32(1.0 / N)
    mu = sq_ref[0:1, :] * inv_n
    var = sq_ref[1:2, :] * inv_n - mu * mu
    scale = lax.rsqrt(var + 1e-5) * gam_ref[...]
    c = cnt_ref[...]
    pooled = p_ref[...] / jnp.maximum(c, 1.0)
    normed = (pooled - mu) * scale + bet_ref[...]
    normed = jnp.where(c > 0.0, normed, 0.0)
    logits = jnp.dot(normed, w_ref[...], preferred_element_type=_F32)
    logits = logits + b_ref[...]
    m = jnp.max(logits, axis=1, keepdims=True)
    sh = logits - m
    out_ref[...] = sh - jnp.log(jnp.sum(jnp.exp(sh), axis=1, keepdims=True))


def _head(sq, p, cnt, gam, bet, w, b):
    return pl.pallas_call(
        _head_body,
        out_shape=jax.ShapeDtypeStruct((G, C), _F32),
    )(sq, p, cnt, gam, bet, w, b)


def kernel(x, edge_index, batch, lin1_w, lin1_b, g0_w1, g0_b1, g0_w2, g0_b2,
           g1_w1, g1_b1, g1_w2, g1_b2, g2_w1, g2_b1, g2_w2, g2_b2,
           bn_gamma, bn_beta, lin2_w, lin2_b):
    pad = _PE - E
    src = jnp.concatenate(
        [edge_index[0].astype(jnp.int32), jnp.zeros((pad,), jnp.int32)])
    dst = jnp.concatenate(
        [edge_index[1].astype(jnp.int32), jnp.full((pad,), N, jnp.int32)])
    src2d = src.reshape(_ROWS, 128)
    dst2d = dst.reshape(_ROWS, 128)
    bat = batch.astype(jnp.int32).reshape(GRID, 1, RB)

    hlo, hhi = _lin1(x, lin1_w, lin1_b.reshape(1, H))

    layers = ((g0_w1, g0_b1, g0_w2, g0_b2),
              (g1_w1, g1_b1, g1_w2, g1_b2))
    for w1, b1, w2, b2 in layers:
        alo, ahi = _sc_aggregate(hlo, hhi, src2d, dst2d)
        hlo, hhi = _mlp(hlo, hhi, alo, ahi, w1, b1.reshape(1, H),
                        w2, b2.reshape(1, H))

    alo, ahi = _sc_aggregate(hlo, hhi, src2d, dst2d)
    sq, p, cnt = _mlp_stats(hlo, hhi, alo, ahi, g2_w1, g2_b1.reshape(1, H),
                            g2_w2, g2_b2.reshape(1, H), bat)

    return _head(sq, p, cnt, bn_gamma.reshape(1, H), bn_beta.reshape(1, H),
                 lin2_w, lin2_b.reshape(1, C))


# speedup vs baseline: 12.1690x; 1.2571x over previous
import jax
import jax.numpy as jnp
import numpy as np
from jax import lax
from jax.experimental import pallas as pl
from jax.experimental.pallas import tpu as pltpu
from jax.experimental.pallas import tpu_sc as plsc

N = 100000
E = 1600000
F_IN = 128
H = 32
C = 10
G = 64
HH = H // 2

RB = 2048
GRID = 49
PR = RB // 8
NP = GRID * RB
PROWS = NP // 8
XR = N * F_IN // 1024

_PE = 784 * 2048
_ROWS = _PE // 128
_TROWS = _ROWS // 16
_GU = 2
_NG = _TROWS // _GU
_FR = 6256
_NPAD = 16 * _FR
_FR_LAST = N - 15 * _FR

_F32 = jnp.float32

_I0 = np.int32(0)


def _im_row(i):
    return (i, _I0)


def _im_fix(i):
    return (_I0, _I0)


def _im_bat(i):
    return (i, _I0, _I0)


def _sc_body(hlo, hhi, src, dst, alo, ahi, acc,
             sidx0, sidx1, sidx2, sidx3, didx0, didx1, didx2, didx3,
             rows0, rows1, rows2, rows3,
             gsem0, gsem1, gsem2, gsem3, ssem0, ssem1, ssem2, ssem3):
    cid = lax.axis_index("c")
    sid = lax.axis_index("s")

    def _zb(i, c):
        rows0[jnp.int32(0), i, :] = jnp.zeros((16,), _F32)
        return c

    lax.fori_loop(jnp.int32(0), jnp.int32(128), _zb, jnp.int32(0))
    z0 = rows0.at[jnp.int32(0)]
    zbase = sid * jnp.int32(_FR)
    for t in range(48):
        pltpu.sync_copy(z0, acc.at[pl.ds(zbase + t * 128, 128)])
    pltpu.sync_copy(z0.at[pl.ds(0, 112)],
                    acc.at[pl.ds(zbase + 48 * 128, 112)])
    plsc.subcore_barrier()

    def _run(hsrc, aout):
        row_base = sid * jnp.int32(_TROWS)
        bufs = ((sidx0, didx0, rows0, gsem0, ssem0),
                (sidx1, didx1, rows1, gsem1, ssem1),
                (sidx2, didx2, rows2, gsem2, ssem2),
                (sidx3, didx3, rows3, gsem3, ssem3))

        def _gather(g, p):
            si, di, rw, gs, _ = bufs[p]
            r0 = row_base + g * jnp.int32(_GU)
            pltpu.sync_copy(src.at[pl.ds(r0, _GU)], si)
            pltpu.sync_copy(dst.at[pl.ds(r0, _GU)], di)
            for j in range(_GU):
                pltpu.async_copy(hsrc.at[si.at[jnp.int32(j)]],
                                 rw.at[jnp.int32(j)], gs)

        def _wait_gather(p):
            si, _, rw, gs, _ = bufs[p]
            for j in range(_GU):
                pltpu.make_async_copy(hsrc.at[si.at[jnp.int32(j)]],
                                      rw.at[jnp.int32(j)], gs).wait()

        def _scatter(p):
            _, di, rw, _, ss = bufs[p]
            for j in range(_GU):
                pltpu.async_copy(rw.at[jnp.int32(j)],
                                 acc.at[di.at[jnp.int32(j)]], ss, add=True)

        def _wait_scatter(p):
            _, di, rw, _, ss = bufs[p]
            for j in range(_GU):
                pltpu.make_async_copy(rw.at[jnp.int32(j)],
                                      acc.at[di.at[jnp.int32(j)]], ss).wait()

        assert _NG % 4 == 0
        _gather(jnp.int32(0), 0)
        _gather(jnp.int32(1), 1)
        last_k = _NG // 4 - 1

        def _step(k, c):
            g = k * jnp.int32(4)
            for u in range(4):
                gu = g + u
                nxt = (u + 2) % 4
                if u < 2:
                    @pl.when(k == 0)
                    def _():
                        _wait_gather(u)
                        _scatter(u)
                        _gather(gu + 2, nxt)

                    @pl.when(k > 0)
                    def _():
                        _wait_scatter(nxt)
                        _gather(gu + 2, nxt)
                        _wait_gather(u)
                        _scatter(u)
                else:
                    @pl.when(k < last_k)
                    def _():
                        _wait_scatter(nxt)
                        _gather(gu + 2, nxt)
                        _wait_gather(u)
                        _scatter(u)

                    @pl.when(k == last_k)
                    def _():
                        _wait_scatter(nxt)
                        _wait_gather(u)
                        _scatter(u)
            return c

        lax.fori_loop(jnp.int32(0), jnp.int32(_NG // 4), _step, jnp.int32(0))
        _wait_scatter(2)
        _wait_scatter(3)
        plsc.subcore_barrier()
        fr0 = sid * jnp.int32(_FR)

        @pl.when(sid < 15)
        def _():
            pltpu.sync_copy(acc.at[pl.ds(fr0, _FR)], aout.at[pl.ds(fr0, _FR)])

        @pl.when(sid == 15)
        def _():
            pltpu.sync_copy(acc.at[pl.ds(fr0, _FR_LAST)],
                            aout.at[pl.ds(fr0, _FR_LAST)])

    @pl.when(cid == 0)
    def _():
        _run(hlo, alo)

    @pl.when(cid == 1)
    def _():
        _run(hhi, ahi)


def _sc_aggregate(h_lo, h_hi, src2d, dst2d):
    mesh = plsc.VectorSubcoreMesh(core_axis_name="c", subcore_axis_name="s")
    kern = pl.kernel(
        _sc_body,
        out_type=(jax.ShapeDtypeStruct((NP, HH), _F32),
                  jax.ShapeDtypeStruct((NP, HH), _F32)),
        mesh=mesh,
        scratch_types=(
            [pltpu.VMEM_SHARED((_NPAD, HH), _F32)]
            + [pltpu.VMEM((_GU, 128), jnp.int32) for _ in range(8)]
            + [pltpu.VMEM((_GU, 128, HH), _F32) for _ in range(4)]
            + [pltpu.SemaphoreType.DMA for _ in range(8)]
        ),
        compiler_params=pltpu.CompilerParams(use_tc_tiling_on_sc=False),
    )
    return kern(h_lo, h_hi, src2d, dst2d)


def _relu(v):
    return jnp.maximum(v, 0.0)


def _lin1_body(x_ref, klo_ref, khi_ref, blo_ref, bhi_ref, lo_ref, hi_ref):
    xp = x_ref[...]
    lo_ref[...] = _relu(
        jnp.dot(xp, klo_ref[...], preferred_element_type=_F32) + blo_ref[...])
    hi_ref[...] = _relu(
        jnp.dot(xp, khi_ref[...], preferred_element_type=_F32) + bhi_ref[...])


def _lin1(xp, klo, khi, blo, bhi):
    return pl.pallas_call(
        _lin1_body,
        grid=(GRID,),
        in_specs=[
            pl.BlockSpec((PR, 1024), _im_row),
            pl.BlockSpec((1024, 128), _im_fix),
            pl.BlockSpec((1024, 128), _im_fix),
            pl.BlockSpec((1, 128), _im_fix),
            pl.BlockSpec((1, 128), _im_fix),
        ],
        out_specs=[
            pl.BlockSpec((PR, 128), _im_row),
            pl.BlockSpec((PR, 128), _im_row),
        ],
        out_shape=[jax.ShapeDtypeStruct((PROWS, 128), _F32)] * 2,
    )(xp, klo, khi, blo, bhi)


def _mlp_core(lo_ref, hi_ref, alo_ref, ahi_ref, k1lo_ref, k1hi_ref, b1_ref,
              k2lo_ref, k2hi_ref, b2lo_ref, b2hi_ref):
    mlo = lo_ref[...] + alo_ref[...]
    mhi = hi_ref[...] + ahi_ref[...]
    t = _relu(jnp.dot(mlo, k1lo_ref[...], preferred_element_type=_F32)
              + jnp.dot(mhi, k1hi_ref[...], preferred_element_type=_F32)
              + b1_ref[...])
    olo = _relu(jnp.dot(t, k2lo_ref[...], preferred_element_type=_F32)
                + b2lo_ref[...])
    ohi = _relu(jnp.dot(t, k2hi_ref[...], preferred_element_type=_F32)
                + b2hi_ref[...])
    return olo, ohi


_MLP_W_SPECS = [
    pl.BlockSpec((128, 256), _im_fix),
    pl.BlockSpec((128, 256), _im_fix),
    pl.BlockSpec((1, 256), _im_fix),
    pl.BlockSpec((256, 128), _im_fix),
    pl.BlockSpec((256, 128), _im_fix),
    pl.BlockSpec((1, 128), _im_fix),
    pl.BlockSpec((1, 128), _im_fix),
]


def _mlp_body(lo_ref, hi_ref, alo_ref, ahi_ref, k1lo_ref, k1hi_ref, b1_ref,
              k2lo_ref, k2hi_ref, b2lo_ref, b2hi_ref, olo_ref, ohi_ref):
    olo, ohi = _mlp_core(lo_ref, hi_ref, alo_ref, ahi_ref, k1lo_ref,
                         k1hi_ref, b1_ref, k2lo_ref, k2hi_ref, b2lo_ref,
                         b2hi_ref)
    olo_ref[...] = olo
    ohi_ref[...] = ohi


def _mlp(lo, hi, alo, ahi, kw):
    node_spec = pl.BlockSpec((PR, 128), _im_row)
    return pl.pallas_call(
        _mlp_body,
        grid=(GRID,),
        in_specs=[node_spec] * 4 + _MLP_W_SPECS,
        out_specs=[node_spec, node_spec],
        out_shape=[jax.ShapeDtypeStruct((PROWS, 128), _F32)] * 2,
    )(lo, hi, alo, ahi, *kw)


def _mlp_stats_body(lo_ref, hi_ref, alo_ref, ahi_ref, k1lo_ref, k1hi_ref,
                    b1_ref, k2lo_ref, k2hi_ref, b2lo_ref, b2hi_ref,
                    bat_ref, fold_ref, sq_ref, p_ref, cnt_ref):
    olo, ohi = _mlp_core(lo_ref, hi_ref, alo_ref, ahi_ref, k1lo_ref,
                         k1hi_ref, b1_ref, k2lo_ref, k2hi_ref, b2lo_ref,
                         b2hi_ref)

    i = pl.program_id(0)
    ri = lax.broadcasted_iota(jnp.int32, (PR, 128), 0)
    li = lax.broadcasted_iota(jnp.int32, (PR, 128), 1)
    slot = i * RB + ri * 8 + li // 16
    olo = jnp.where(slot < N, olo, 0.0)
    ohi = jnp.where(slot < N, ohi, 0.0)

    @pl.when(i == 0)
    def _():
        sq_ref[...] = jnp.zeros_like(sq_ref)
        p_ref[...] = jnp.zeros_like(p_ref)
        cnt_ref[...] = jnp.zeros_like(cnt_ref)

    fold = fold_ref[...]
    sq_ref[0:1, :HH] += jnp.dot(jnp.sum(olo, 0, keepdims=True), fold,
                                preferred_element_type=_F32)
    sq_ref[0:1, HH:] += jnp.dot(jnp.sum(ohi, 0, keepdims=True), fold,
                                preferred_element_type=_F32)
    sq_ref[1:2, :HH] += jnp.dot(jnp.sum(olo * olo, 0, keepdims=True), fold,
                                preferred_element_type=_F32)
    sq_ref[1:2, HH:] += jnp.dot(jnp.sum(ohi * ohi, 0, keepdims=True), fold,
                                preferred_element_type=_F32)

    bat = bat_ref[...].reshape(PR, 8)
    giota = lax.broadcasted_iota(jnp.int32, (1, G), 1)
    ones_col = jnp.ones((PR, 1), _F32)
    dn = (((0,), (0,)), ((), ()))
    for k in range(8):
        oh = (bat[:, k:k + 1] == giota).astype(_F32)
        p_ref[:, :HH] += lax.dot_general(oh, olo[:, 16 * k:16 * (k + 1)], dn,
                                         preferred_element_type=_F32)
        p_ref[:, HH:] += lax.dot_general(oh, ohi[:, 16 * k:16 * (k + 1)], dn,
                                         preferred_element_type=_F32)
        cnt_ref[...] += lax.dot_general(oh, ones_col, dn,
                                        preferred_element_type=_F32)


def _mlp_stats(lo, hi, alo, ahi, kw, bat, fold):
    node_spec = pl.BlockSpec((PR, 128), _im_row)
    return pl.pallas_call(
        _mlp_stats_body,
        grid=(GRID,),
        in_specs=[node_spec] * 4 + _MLP_W_SPECS + [
            pl.BlockSpec((1, PR, 8), _im_bat),
            pl.BlockSpec((128, HH), _im_fix),
        ],
        out_specs=[
            pl.BlockSpec((2, H), _im_fix),
            pl.BlockSpec((G, H), _im_fix),
            pl.BlockSpec((G, 1), _im_fix),
        ],
        out_shape=[
            jax.ShapeDtypeStruct((2, H), _F32),
            jax.ShapeDtypeStruct((G, H), _F32),
            jax.ShapeDtypeStruct((G, 1), _F32),
        ],
    )(lo, hi, alo, ahi, *kw, bat, fold)


def _head_body(sq_ref, p_ref, cnt_ref, gam_ref, bet_ref, w_ref, b_ref,
               out_ref):
    inv_n = _F32(1.0 / N)
    mu = sq_ref[0:1, :] * inv_n
    var = sq_ref[1:2, :] * inv_n - mu * mu
    scale = lax.rsqrt(var + 1e-5) * gam_ref[...]
    c = cnt_ref[...]
    pooled = p_ref[...] / jnp.maximum(c, 1.0)
    normed = (pooled - mu) * scale + bet_ref[...]
    normed = jnp.where(c > 0.0, normed, 0.0)
    logits = jnp.dot(normed, w_ref[...], preferred_element_type=_F32)
    logits = logits + b_ref[...]
    m = jnp.max(logits, axis=1, keepdims=True)
    sh = logits - m
    out_ref[...] = sh - jnp.log(jnp.sum(jnp.exp(sh), axis=1, keepdims=True))


def _head(sq, p, cnt, gam, bet, w, b):
    return pl.pallas_call(
        _head_body,
        out_shape=jax.ShapeDtypeStruct((G, C), _F32),
    )(sq, p, cnt, gam, bet, w, b)


def _kron8(w):
    return jnp.kron(jnp.eye(8, dtype=_F32), w)


def _tile8(v):
    return jnp.tile(v.reshape(1, -1), (1, 8))


def kernel(x, edge_index, batch, lin1_w, lin1_b, g0_w1, g0_b1, g0_w2, g0_b2,
           g1_w1, g1_b1, g1_w2, g1_b2, g2_w1, g2_b1, g2_w2, g2_b2,
           bn_gamma, bn_beta, lin2_w, lin2_b):
    pad = _PE - E
    src = jnp.concatenate(
        [edge_index[0].astype(jnp.int32), jnp.zeros((pad,), jnp.int32)])
    dst = jnp.concatenate(
        [edge_index[1].astype(jnp.int32), jnp.full((pad,), N, jnp.int32)])
    src2d = src.reshape(_ROWS, 128)
    dst2d = dst.reshape(_ROWS, 128)
    bat = jnp.concatenate(
        [batch.astype(jnp.int32), jnp.full((NP - N,), 127, jnp.int32)]
    ).reshape(GRID, PR, 8)
    fold = jnp.tile(jnp.eye(HH, dtype=_F32), (8, 1))

    xp = x.reshape(XR, 1024)
    hlo, hhi = _lin1(xp, _kron8(lin1_w[:, :HH]), _kron8(lin1_w[:, HH:]),
                     _tile8(lin1_b[:HH]), _tile8(lin1_b[HH:]))

    def packed_weights(w1, b1, w2, b2):
        return (_kron8(w1[:HH, :]), _kron8(w1[HH:, :]), _tile8(b1),
                _kron8(w2[:, :HH]), _kron8(w2[:, HH:]),
                _tile8(b2[:HH]), _tile8(b2[HH:]))

    for w1, b1, w2, b2 in ((g0_w1, g0_b1, g0_w2, g0_b2),
                           (g1_w1, g1_b1, g1_w2, g1_b2)):
        alo, ahi = _sc_aggregate(hlo.reshape(NP, HH), hhi.reshape(NP, HH),
                                 src2d, dst2d)
        hlo, hhi = _mlp(hlo, hhi, alo.reshape(PROWS, 128),
                        ahi.reshape(PROWS, 128),
                        packed_weights(w1, b1, w2, b2))

    alo, ahi = _sc_aggregate(hlo.reshape(NP, HH), hhi.reshape(NP, HH),
                             src2d, dst2d)
    sq, p, cnt = _mlp_stats(hlo, hhi, alo.reshape(PROWS, 128),
                            ahi.reshape(PROWS, 128),
                            packed_weights(g2_w1, g2_b1, g2_w2, g2_b2),
                            bat, fold)

    return _head(sq, p, cnt, bn_gamma.reshape(1, H), bn_beta.reshape(1, H),
                 lin2_w, lin2_b.reshape(1, C))
